# baseline (speedup 1.0000x reference)
import functools

import jax
import jax.numpy as jnp
from jax import lax
from jax.experimental import pallas as pl
from jax.experimental.pallas import tpu as pltpu
from jax.experimental.pallas import tpu_sc as plsc

NUM_EMBEDDINGS = 1000000
EMBED_DIM = 64
BATCH = 4096
SEQ = 200
EPS = 1e-5

LANES = 16
NWORK = 32
PAIR_ROWS = NUM_EMBEDDINGS // 2
N_FULL_BLK = NUM_EMBEDDINGS // 128
TAIL_IDS = NUM_EMBEDDINGS - N_FULL_BLK * 128
BBLK = BATCH // NWORK
SBW = 256
N_SB = (N_FULL_BLK * 128) // SBW
DEPTH = 4

_params = pltpu.CompilerParams(
    use_tc_tiling_on_sc=True, needs_layout_passes=False)


def _iota16():
    return lax.iota(jnp.int32, LANES)


def _splat(x):
    return jnp.full((LANES,), x, jnp.int32)


def _fast_rsqrt(w):
    i = lax.bitcast_convert_type(w, jnp.int32)
    i = jnp.int32(0x5F3759DF) - lax.shift_right_logical(i, 1)
    y = lax.bitcast_convert_type(i, jnp.float32)
    half = jnp.float32(0.5) * w
    for _ in range(3):
        y = y * (jnp.float32(1.5) - half * y * y)
    return y


def _make_phase1():
    mesh = plsc.VectorSubcoreMesh(core_axis_name="c", subcore_axis_name="s")

    @functools.partial(
        pl.kernel,
        mesh=mesh,
        out_type=jax.ShapeDtypeStruct((PAIR_ROWS, 128), jnp.float32),
        scratch_types=[
            pltpu.VMEM((EMBED_DIM, SBW), jnp.float32),
            pltpu.VMEM((EMBED_DIM, SBW), jnp.float32),
            pltpu.VMEM((SBW // 2, 128), jnp.float32),
            pltpu.VMEM((SBW // 2, 128), jnp.float32),
            pltpu.SemaphoreType.DMA,
            pltpu.SemaphoreType.DMA,
            pltpu.SemaphoreType.DMA,
            pltpu.SemaphoreType.DMA,
        ],
        compiler_params=_params,
    )
    def phase1(tab_t_hbm, tail_hbm, scratch_hbm,
               slab_a, slab_b, buf_a, buf_b, isem_a, isem_b, osem_a, osem_b):
        wid = lax.axis_index("s") * 2 + lax.axis_index("c")
        it16 = _iota16()
        row_idx = [k * LANES + it16 for k in range(4)]

        def start_in(sb, slab, isem):
            c0 = pl.multiple_of(sb * SBW, 128)
            pltpu.async_copy(tab_t_hbm.at[:, pl.ds(c0, SBW)], slab, isem)

        def wait_in(slab, isem):
            pltpu.make_async_copy(
                tab_t_hbm.at[:, pl.ds(0, SBW)], slab, isem).wait()

        def drain_out(buf, osem):
            pltpu.make_async_copy(
                buf, scratch_hbm.at[pl.ds(0, SBW // 2), :], osem).wait()

        def do_sb(i, sb, slab, isem, buf, osem, nxt_slab, nxt_isem):
            @pl.when(sb < N_SB)
            def _():
                wait_in(slab, isem)

                @pl.when(sb + NWORK < N_SB)
                def _():
                    start_in(sb + NWORK, nxt_slab, nxt_isem)

                @pl.when(i >= 2)
                def _():
                    drain_out(buf, osem)

                def j_body(jj, jc):
                    for u in range(2):
                        j = jj * 2 + u
                        for h in range(2):
                            col = _splat(2 * j + h)
                            for k in range(4):
                                vals = plsc.load_gather(
                                    slab, [row_idx[k], col])
                                buf[j, pl.ds(h * 64 + k * LANES, LANES)] = \
                                    vals
                    return jc

                lax.fori_loop(0, SBW // 4, j_body, 0)
                r0 = pl.multiple_of(sb * (SBW // 2), 64)
                pltpu.async_copy(
                    buf, scratch_hbm.at[pl.ds(r0, SBW // 2), :], osem)

        start_in(wid, slab_a, isem_a)

        def pair_body(t, carry):
            i = t * 2
            do_sb(i, wid + i * NWORK, slab_a, isem_a, buf_a, osem_a,
                  slab_b, isem_b)
            do_sb(i + 1, wid + (i + 1) * NWORK, slab_b, isem_b, buf_b, osem_b,
                  slab_a, isem_a)
            return carry

        n_iter = (N_SB + NWORK - 1) // NWORK
        lax.fori_loop(0, (n_iter + 1) // 2, pair_body, 0)
        drain_out(buf_a, osem_a)
        drain_out(buf_b, osem_b)

        @pl.when(wid == NWORK - 1)
        def _tail():
            pltpu.sync_copy(tail_hbm, slab_a.at[:, pl.ds(0, 128)])

            def j_body(j, jc):
                for h in range(2):
                    col = _splat(2 * j + h)
                    for k in range(4):
                        vals = plsc.load_gather(slab_a, [row_idx[k], col])
                        buf_a[j, pl.ds(h * 64 + k * LANES, LANES)] = vals
                return jc

            lax.fori_loop(0, TAIL_IDS // 2, j_body, 0)
            pltpu.sync_copy(
                buf_a.at[pl.ds(0, TAIL_IDS // 2), :],
                scratch_hbm.at[pl.ds(N_FULL_BLK * 64, TAIL_IDS // 2), :])

    return phase1


def _make_phase2():
    mesh = plsc.VectorSubcoreMesh(core_axis_name="c", subcore_axis_name="s")

    @functools.partial(
        pl.kernel,
        mesh=mesh,
        out_type=jax.ShapeDtypeStruct((SEQ, EMBED_DIM, BATCH), jnp.float32),
        scratch_types=[
            pltpu.VMEM((SEQ, BBLK), jnp.int32),
            [pltpu.VMEM((BBLK,), jnp.int32) for _ in range(DEPTH)],
            [pltpu.VMEM((BBLK, 128), jnp.float32) for _ in range(DEPTH)],
            pltpu.VMEM((EMBED_DIM, BBLK), jnp.float32),
            pltpu.VMEM((EMBED_DIM, BBLK), jnp.float32),
            pltpu.VMEM((EMBED_DIM, LANES), jnp.float32),
            pltpu.VMEM((EMBED_DIM, LANES), jnp.float32),
            pltpu.VMEM((EMBED_DIM,), jnp.float32),
            pltpu.VMEM((EMBED_DIM,), jnp.float32),
            [pltpu.SemaphoreType.DMA for _ in range(DEPTH)],
            pltpu.SemaphoreType.DMA,
            pltpu.SemaphoreType.DMA,
        ],
        compiler_params=_params,
    )
    def phase2(scratch_hbm, xt_hbm, gamma_hbm, beta_hbm, out_hbm,
               xblk_v, idx2_l, rows_l, slab0_v, slab1_v,
               gs_v, bs_v, gtmp_v, btmp_v, sem_l, osem0, osem1):
        wid = lax.axis_index("s") * 2 + lax.axis_index("c")
        it16 = _iota16()
        row_idx = [g * LANES + it16 for g in range(8)]
        inv_d = jnp.float32(1.0 / EMBED_DIM)

        pltpu.sync_copy(xt_hbm.at[:, pl.ds(wid * BBLK, BBLK)], xblk_v)
        pltpu.sync_copy(gamma_hbm, gtmp_v)
        pltpu.sync_copy(beta_hbm, btmp_v)

        def gb_body(d, c):
            gs_v[d, pl.ds(0, LANES)] = plsc.load_gather(gtmp_v, [_splat(d)])
            bs_v[d, pl.ds(0, LANES)] = plsc.load_gather(btmp_v, [_splat(d)])
            return c

        lax.fori_loop(0, EMBED_DIM, gb_body, 0)

        def start_gather(s, k):
            for g in range(8):
                xv = xblk_v[s, pl.ds(g * LANES, LANES)]
                idx2_l[k][pl.ds(g * LANES, LANES)] = \
                    lax.shift_right_logical(xv, 1)
            pltpu.async_copy(scratch_hbm.at[idx2_l[k]], rows_l[k], sem_l[k])

        def wait_gather(k):
            pltpu.make_async_copy(
                scratch_hbm.at[pl.ds(0, BBLK)], rows_l[k], sem_l[k]).wait()

        def drain_out(slab_v, sem):
            pltpu.make_async_copy(
                slab_v, out_hbm.at[0, :, pl.ds(0, BBLK)], sem).wait()

        def compute_item(s, rows_v, slab_v, osem):
            par64 = []
            for g in range(8):
                xv = xblk_v[s, pl.ds(g * LANES, LANES)]
                par64.append((xv & 1) * 64)

            def d_body(dd, acc):
                acc = list(acc)
                for u in range(4):
                    d = dd * 4 + u
                    for g in range(8):
                        v = plsc.load_gather(
                            rows_v, [row_idx[g], par64[g] + d])
                        acc[g] = acc[g] + v
                        acc[8 + g] = acc[8 + g] + v * v
                return tuple(acc)

            zero = jnp.zeros((LANES,), jnp.float32)
            acc = lax.fori_loop(0, EMBED_DIM // 4, d_body, (zero,) * 16)
            mean, rstd = [], []
            for g in range(8):
                m = acc[g] * inv_d
                var = acc[8 + g] * inv_d - m * m
                mean.append(m)
                rstd.append(_fast_rsqrt(var + jnp.float32(EPS)))

            @pl.when(s >= 2)
            def _():
                drain_out(slab_v, osem)

            def d2_body(dd, c):
                for u in range(4):
                    d = dd * 4 + u
                    gsd = gs_v[d, pl.ds(0, LANES)]
                    bsd = bs_v[d, pl.ds(0, LANES)]
                    for g in range(8):
                        v = plsc.load_gather(
                            rows_v, [row_idx[g], par64[g] + d])
                        o = (v - mean[g]) * rstd[g] * gsd + bsd
                        slab_v[d, pl.ds(g * LANES, LANES)] = o
                return c

            lax.fori_loop(0, EMBED_DIM // 4, d2_body, 0)
            pltpu.async_copy(
                slab_v, out_hbm.at[s, :, pl.ds(wid * BBLK, BBLK)], osem)

        for k in range(DEPTH):
            start_gather(k, k)

        slabs = (slab0_v, slab1_v)
        osems = (osem0, osem1)

        def ring_body(s4, carry):
            for u in range(DEPTH):
                s = s4 * DEPTH + u
                wait_gather(u)
                compute_item(s, rows_l[u], slabs[u % 2], osems[u % 2])

                @pl.when(s + DEPTH < SEQ)
                def _():
                    start_gather(s + DEPTH, u)
            return carry

        lax.fori_loop(0, SEQ // DEPTH, ring_body, 0)
        drain_out(slab0_v, osem0)
        drain_out(slab1_v, osem1)

    return phase2


_phase1 = _make_phase1()
_phase2 = _make_phase2()


@jax.jit
def kernel(x, table, gamma, beta):
    tail = jnp.pad(
        jnp.transpose(lax.slice(table, (N_FULL_BLK * 128, 0),
                                (NUM_EMBEDDINGS, EMBED_DIM))),
        ((0, 0), (0, 128 - TAIL_IDS)))
    scratch = _phase1(table.T, tail)
    out_t = _phase2(scratch, x.T, gamma, beta)
    return jnp.transpose(out_t, (2, 0, 1))
